# baseline (speedup 1.0000x reference)
import functools

import jax
import jax.numpy as jnp
from jax import lax
from jax.experimental import pallas as pl
from jax.experimental.pallas import tpu as pltpu
from jax.experimental.pallas import tpu_sc as plsc

_EMB = 32
_L = 16
_NC = 2
_NS = 16
_NW = _NC * _NS
_CH = 128


def _make_gather(batch: int, hist: int):
    k_vec = _CH // _L
    n_pairs = hist // 2

    mesh = plsc.VectorSubcoreMesh(core_axis_name="c", subcore_axis_name="s")

    @functools.partial(
        pl.kernel,
        mesh=mesh,
        compiler_params=pltpu.CompilerParams(
            use_tc_tiling_on_sc=False, needs_layout_passes=False),
        out_type=jax.ShapeDtypeStruct((hist, _EMB, batch), jnp.float32),
        scratch_types=[
            pltpu.VMEM((hist, _CH), jnp.int32),
            pltpu.VMEM((2, _EMB, _CH), jnp.float32),
            pltpu.SemaphoreType.DMA,
            pltpu.SemaphoreType.DMA,
            pltpu.SemaphoreType.DMA,
            pltpu.SemaphoreType.DMA,
        ],
    )
    def gather_kernel(xt_hbm, tablet_hbm, out_hbm, idx_v, tbuf_v,
                      gsem0, gsem1, wsem0, wsem1):
        gsems = (gsem0, gsem1)
        wsems = (wsem0, wsem1)
        wid = lax.axis_index("s") * _NC + lax.axis_index("c")
        base_b = wid * _CH
        pltpu.sync_copy(xt_hbm.at[:, pl.ds(base_b, _CH)], idx_v)

        def prep(h):
            for k in range(k_vec):
                sl = pl.ds(k * _L, _L)
                idx_v[h, sl] = idx_v[h, sl] + 1

        def fire(h, buf):
            def plane(c, carry):
                pltpu.async_copy(
                    tablet_hbm.at[c].at[idx_v.at[h]],
                    tbuf_v.at[buf, c],
                    gsems[buf])
                return carry

            lax.fori_loop(0, _EMB, plane, 0)

        def wait_gather(buf):
            pltpu.make_async_copy(
                tablet_hbm.at[:, pl.ds(0, _CH)], tbuf_v.at[buf],
                gsems[buf]).wait()

        def write(h, buf):
            pltpu.async_copy(
                tbuf_v.at[buf],
                out_hbm.at[h, :, pl.ds(base_b, _CH)],
                wsems[buf])

        def wait_write(buf):
            pltpu.make_async_copy(
                tbuf_v.at[buf], out_hbm.at[0, :, pl.ds(base_b, _CH)],
                wsems[buf]).wait()

        prep(0)
        fire(0, 0)
        prep(1)
        fire(1, 1)

        def outer(o, carry):
            h0 = 2 * o
            wait_gather(0)
            write(h0, 0)

            @pl.when(o < n_pairs - 1)
            def _():
                prep(h0 + 2)
                wait_write(0)
                fire(h0 + 2, 0)

            wait_gather(1)
            write(h0 + 1, 1)

            @pl.when(o < n_pairs - 1)
            def _():
                prep(h0 + 3)
                wait_write(1)
                fire(h0 + 3, 1)

            return carry

        lax.fori_loop(0, n_pairs, outer, 0)
        wait_write(0)
        wait_write(1)

    return gather_kernel


def kernel(x, emb_usr):
    batch, hist = x.shape
    xt = x.T
    tablet = emb_usr.T
    out_t = _make_gather(batch, hist)(xt, tablet)
    return out_t.transpose(2, 0, 1)


# speedup vs baseline: 4.5429x; 4.5429x over previous
import functools

import jax
import jax.numpy as jnp
from jax import lax
from jax.experimental import pallas as pl
from jax.experimental.pallas import tpu as pltpu
from jax.experimental.pallas import tpu_sc as plsc

_EMB = 32
_L = 16
_NC = 2
_NS = 16
_NW = _NC * _NS
_CH = 128


def _make_gather(batch: int, hist: int):
    k_vec = _CH // _L
    n_pairs = hist // 2

    mesh = plsc.VectorSubcoreMesh(core_axis_name="c", subcore_axis_name="s")

    @functools.partial(
        pl.kernel,
        mesh=mesh,
        compiler_params=pltpu.CompilerParams(
            use_tc_tiling_on_sc=False, needs_layout_passes=False),
        out_type=jax.ShapeDtypeStruct((hist, batch, _EMB), jnp.float32),
        scratch_types=[
            pltpu.VMEM((hist, _CH), jnp.int32),
            pltpu.VMEM((2, _CH, _EMB), jnp.float32),
            pltpu.SemaphoreType.DMA,
            pltpu.SemaphoreType.DMA,
            pltpu.SemaphoreType.DMA,
            pltpu.SemaphoreType.DMA,
        ],
    )
    def gather_kernel(xt_hbm, table_hbm, out_hbm, idx_v, rows_v,
                      gsem0, gsem1, wsem0, wsem1):
        gsems = (gsem0, gsem1)
        wsems = (wsem0, wsem1)
        wid = lax.axis_index("s") * _NC + lax.axis_index("c")
        base_b = wid * _CH
        pltpu.sync_copy(xt_hbm.at[:, pl.ds(base_b, _CH)], idx_v)

        def prep(h):
            for k in range(k_vec):
                sl = pl.ds(k * _L, _L)
                idx_v[h, sl] = idx_v[h, sl] + 1

        def fire(h, buf):
            pltpu.async_copy(
                table_hbm.at[idx_v.at[h]], rows_v.at[buf], gsems[buf])

        def wait_gather(buf):
            pltpu.make_async_copy(
                table_hbm.at[pl.ds(0, _CH)], rows_v.at[buf],
                gsems[buf]).wait()

        def write(h, buf):
            pltpu.async_copy(
                rows_v.at[buf],
                out_hbm.at[h, pl.ds(base_b, _CH)],
                wsems[buf])

        def wait_write(buf):
            pltpu.make_async_copy(
                rows_v.at[buf], out_hbm.at[0, pl.ds(base_b, _CH)],
                wsems[buf]).wait()

        prep(0)
        fire(0, 0)
        prep(1)
        fire(1, 1)

        def outer(o, carry):
            h0 = 2 * o
            wait_gather(0)
            write(h0, 0)

            @pl.when(o < n_pairs - 1)
            def _():
                prep(h0 + 2)
                wait_write(0)
                fire(h0 + 2, 0)

            wait_gather(1)
            write(h0 + 1, 1)

            @pl.when(o < n_pairs - 1)
            def _():
                prep(h0 + 3)
                wait_write(1)
                fire(h0 + 3, 1)

            return carry

        lax.fori_loop(0, n_pairs, outer, 0)
        wait_write(0)
        wait_write(1)

    return gather_kernel


def kernel(x, emb_usr):
    batch, hist = x.shape
    xt = x.T
    out_t = _make_gather(batch, hist)(xt, emb_usr)
    return out_t.transpose(1, 0, 2)
